# baseline (speedup 1.0000x reference)
import functools

import jax
import jax.numpy as jnp
from jax import lax
from jax.experimental import pallas as pl
from jax.experimental.pallas import tpu as pltpu
from jax.experimental.pallas import tpu_sc as plsc

_VOCAB = 100000
_MAX_LEN = 2048
_DIM = 1024
_BATCH = 4

_NC = 2
_NS = 16
_NW = _NC * _NS
_T_PER_W = _MAX_LEN // _NW
_CHUNK = 16
_NCHUNK = _BATCH * _T_PER_W // _CHUNK
_NH = _T_PER_W // _CHUNK
_LANES = 16
_NBUF = 5
_GDEPTH = 3
_UNROLL = 2

_mesh = plsc.VectorSubcoreMesh(core_axis_name="c", subcore_axis_name="s")


@functools.partial(
    pl.kernel,
    mesh=_mesh,
    out_type=jax.ShapeDtypeStruct((_BATCH, _MAX_LEN, _DIM), jnp.float32),
    scratch_types=(
        [pltpu.VMEM((_BATCH * _T_PER_W,), jnp.int32)]
        + [pltpu.VMEM((_CHUNK, _DIM), jnp.float32) for _ in range(2)]
        + [pltpu.VMEM((_CHUNK, _DIM), jnp.float32) for _ in range(_NBUF)]
        + [pltpu.SemaphoreType.DMA for _ in range(3 + 2 * _NBUF)]
    ),
)
def _embed(idx_hbm, table_hbm, pos_hbm, out_hbm, idx_v, *scratch):
    pring = scratch[:2]
    bufs = scratch[2:2 + _NBUF]
    psems = scratch[2 + _NBUF:4 + _NBUF]
    isem = scratch[4 + _NBUF]
    gsems = scratch[5 + _NBUF:5 + 2 * _NBUF]
    wsems = scratch[5 + 2 * _NBUF:5 + 3 * _NBUF]

    wid = lax.axis_index("s") * _NC + lax.axis_index("c")
    t0 = wid * _T_PER_W

    idx_handles = [
        pltpu.async_copy(idx_hbm.at[b, pl.ds(t0, _T_PER_W)],
                         idx_v.at[pl.ds(b * _T_PER_W, _T_PER_W)], isem)
        for b in range(_BATCH)
    ]

    def pos_load(h):
        return pltpu.async_copy(
            pos_hbm.at[pl.ds(t0 + h * _CHUNK, _CHUNK)],
            pring[h % 2], psems[h % 2])

    def gather(c):
        h, b = divmod(c, _BATCH)
        return pltpu.async_copy(
            table_hbm.at[idx_v.at[pl.ds(b * _T_PER_W + h * _CHUNK, _CHUNK)]],
            bufs[c % _NBUF], gsems[c % _NBUF])

    def writeback(c):
        h, b = divmod(c, _BATCH)
        return pltpu.async_copy(
            bufs[c % _NBUF],
            out_hbm.at[b, pl.ds(t0 + h * _CHUNK, _CHUNK)],
            wsems[c % _NBUF])

    hp = [pos_load(0), pos_load(1)]
    for hnd in idx_handles:
        hnd.wait()
    pos_ready = [False, False]
    hw = [None] * _NBUF
    hg = [None] * _NBUF
    issued = 0
    for c in range(_NCHUNK):
        h = c // _BATCH
        while issued < min(c + 1 + _GDEPTH, _NCHUNK):
            slot = issued % _NBUF
            if hw[slot] is not None:
                hw[slot].wait()
                hw[slot] = None
            hg[slot] = gather(issued)
            issued += 1
        hg[c % _NBUF].wait()
        if not pos_ready[h % 2]:
            hp[h % 2].wait()
            pos_ready[h % 2] = True
        buf = bufs[c % _NBUF]
        pos = pring[h % 2]

        def add_rows(i, _, buf=buf, pos=pos):
            r0 = i * _UNROLL
            for dr in range(_UNROLL):
                for cc in range(_DIM // _LANES):
                    sl = pl.ds(cc * _LANES, _LANES)
                    plsc.addupdate(buf.at[r0 + dr, sl], pos[r0 + dr, sl])
            return 0

        lax.fori_loop(0, _CHUNK // _UNROLL, add_rows, 0)
        if c % _BATCH == _BATCH - 1:
            pos_ready[h % 2] = False
            if h + 2 <= _NH - 1:
                hp[h % 2] = pos_load(h + 2)
        hw[c % _NBUF] = writeback(c)
    for hnd in hw:
        if hnd is not None:
            hnd.wait()


def kernel(inputs, token_table, pos_table):
    return _embed(inputs.astype(jnp.int32), token_table, pos_table)
